# baseline (speedup 1.0000x reference)
import functools

import jax
import jax.numpy as jnp
from jax import lax
from jax.experimental import pallas as pl
from jax.experimental.pallas import tpu as pltpu
from jax.experimental.pallas import tpu_sc as plsc

B = 16384
D = 64
L = 16
NC = 2
NS = 16
NW = NC * NS
BPW = B // NW
CHUNK = 128
NCHUNK = BPW // CHUNK

_mesh = plsc.VectorSubcoreMesh(core_axis_name="c", subcore_axis_name="s")


@functools.partial(
    pl.kernel,
    mesh=_mesh,
    out_type=jax.ShapeDtypeStruct((B,), jnp.float32),
    compiler_params=pltpu.CompilerParams(
        needs_layout_passes=False, use_tc_tiling_on_sc=False),
    scratch_types=[
        pltpu.VMEM((NCHUNK, CHUNK), jnp.int32),
        pltpu.VMEM((NCHUNK, CHUNK), jnp.int32),
        pltpu.VMEM((BPW, D), jnp.float32),
        pltpu.VMEM((BPW, D), jnp.float32),
        pltpu.VMEM((BPW,), jnp.float32),
        pltpu.VMEM((BPW,), jnp.float32),
        pltpu.VMEM((BPW,), jnp.float32),
        pltpu.VMEM((L,), jnp.float32),
        pltpu.SemaphoreType.DMA,
    ],
)
def _cf_sc_kernel(u_ids_hbm, s_ids_hbm, u_embed_hbm, s_embed_hbm,
                  u_bias_hbm, s_bias_hbm, gb_hbm, out_hbm,
                  uidx_v, sidx_v, urows_v, srows_v, ub_v, sb_v, out_v,
                  gb_v, sem):
    wid = lax.axis_index("s") * NC + lax.axis_index("c")
    row0 = wid * NCHUNK

    pltpu.sync_copy(u_ids_hbm.at[pl.ds(row0, NCHUNK)], uidx_v)
    pltpu.sync_copy(s_ids_hbm.at[pl.ds(row0, NCHUNK)], sidx_v)
    pltpu.sync_copy(gb_hbm, gb_v)

    copies = []
    for j in range(NCHUNK):
        dst = pl.ds(j * CHUNK, CHUNK)
        copies.append(pltpu.async_copy(
            u_embed_hbm.at[uidx_v.at[j]], urows_v.at[dst], sem))
        copies.append(pltpu.async_copy(
            s_embed_hbm.at[sidx_v.at[j]], srows_v.at[dst], sem))
        copies.append(pltpu.async_copy(
            u_bias_hbm.at[uidx_v.at[j]], ub_v.at[dst], sem))
        copies.append(pltpu.async_copy(
            s_bias_hbm.at[sidx_v.at[j]], sb_v.at[dst], sem))
    for c in copies:
        c.wait()

    gb = gb_v[...]

    def group(g, carry):
        base = g * L
        rows = base + lax.iota(jnp.int32, L)
        acc = jnp.zeros((L,), jnp.float32)
        for d in range(D):
            col = jnp.full((L,), d, jnp.int32)
            uv = plsc.load_gather(urows_v, [rows, col])
            sv = plsc.load_gather(srows_v, [rows, col])
            acc = acc + uv * sv
        logits = acc + ub_v[pl.ds(base, L)] + sb_v[pl.ds(base, L)] + gb
        out_v[pl.ds(base, L)] = 1.0 / (1.0 + jnp.exp(-logits))
        return carry

    lax.fori_loop(0, BPW // L, group, 0)

    pltpu.sync_copy(out_v, out_hbm.at[pl.ds(wid * BPW, BPW)])


def kernel(u_ids, s_ids, u_embed, s_embed, u_bias, s_bias, global_bias):
    u2 = u_ids.reshape(B // CHUNK, CHUNK)
    s2 = s_ids.reshape(B // CHUNK, CHUNK)
    gb = jnp.broadcast_to(global_bias, (L,))
    return _cf_sc_kernel(u2, s2, u_embed, s_embed,
                         u_bias.reshape(-1), s_bias.reshape(-1), gb)


# speedup vs baseline: 1.4407x; 1.4407x over previous
import functools

import jax
import jax.numpy as jnp
from jax import lax
from jax.experimental import pallas as pl
from jax.experimental.pallas import tpu as pltpu
from jax.experimental.pallas import tpu_sc as plsc

B = 16384
D = 64
L = 16
NC = 2
NS = 16
NW = NC * NS
BPW = B // NW
CHUNK = 128
NCHUNK = BPW // CHUNK
PB = 128
NPHASE = BPW // PB

_mesh = plsc.VectorSubcoreMesh(core_axis_name="c", subcore_axis_name="s")


@functools.partial(
    pl.kernel,
    mesh=_mesh,
    out_type=jax.ShapeDtypeStruct((B,), jnp.float32),
    compiler_params=pltpu.CompilerParams(needs_layout_passes=False),
    scratch_types=[
        pltpu.VMEM((BPW,), jnp.int32),
        pltpu.VMEM((BPW,), jnp.int32),
        pltpu.VMEM((PB, D), jnp.float32),
        pltpu.VMEM((PB, D), jnp.float32),
        pltpu.VMEM((PB, D), jnp.float32),
        pltpu.VMEM((PB, D), jnp.float32),
        pltpu.VMEM((BPW,), jnp.float32),
        pltpu.VMEM((BPW,), jnp.float32),
        pltpu.VMEM((BPW,), jnp.float32),
        pltpu.VMEM((L,), jnp.float32),
        pltpu.SemaphoreType.DMA,
        pltpu.SemaphoreType.DMA,
        pltpu.SemaphoreType.DMA,
    ],
)
def _cf_sc_kernel(u_ids_hbm, s_ids_hbm, u_embed_hbm, s_embed_hbm,
                  u_bias_hbm, s_bias_hbm, gb_hbm, out_hbm,
                  uidx_v, sidx_v, urows0, urows1,
                  srows0, srows1, ub_v, sb_v, out_v, gb_v,
                  sem0, sem1, sem_bias):
    wid = lax.axis_index("s") * NC + lax.axis_index("c")
    base = wid * BPW

    pltpu.sync_copy(u_ids_hbm.at[pl.ds(base, BPW)], uidx_v)
    pltpu.sync_copy(s_ids_hbm.at[pl.ds(base, BPW)], sidx_v)
    pltpu.sync_copy(gb_hbm, gb_v)

    bias_copies = []
    for j in range(NCHUNK):
        dst = pl.ds(j * CHUNK, CHUNK)
        bias_copies.append(pltpu.async_copy(
            u_bias_hbm.at[uidx_v.at[dst]], ub_v.at[dst], sem_bias))
        bias_copies.append(pltpu.async_copy(
            s_bias_hbm.at[sidx_v.at[dst]], sb_v.at[dst], sem_bias))

    def fire_phase(p, ub, sb, sem):
        def body(g, carry):
            uvec = uidx_v[pl.ds(p * PB + g * L, L)]
            svec = sidx_v[pl.ds(p * PB + g * L, L)]
            for k in range(L):
                uid = uvec[k]
                sid = svec[k]
                i = g * L + k
                pltpu.async_copy(u_embed_hbm.at[pl.ds(uid, 1), :],
                                 ub.at[pl.ds(i, 1), :], sem)
                pltpu.async_copy(s_embed_hbm.at[pl.ds(sid, 1), :],
                                 sb.at[pl.ds(i, 1), :], sem)
            return carry
        lax.fori_loop(0, PB // L, body, 0)

    def drain_phase(ub, sb, sem):
        pltpu.make_async_copy(u_embed_hbm.at[pl.ds(0, PB), :], ub, sem).wait()
        pltpu.make_async_copy(u_embed_hbm.at[pl.ds(0, PB), :], sb, sem).wait()

    def compute_phase(p, ub, sb, gb):
        def group(g, carry):
            rows = g * L + lax.iota(jnp.int32, L)
            acc = jnp.zeros((L,), jnp.float32)
            for d in range(D):
                col = jnp.full((L,), d, jnp.int32)
                uv = plsc.load_gather(ub, [rows, col])
                sv = plsc.load_gather(sb, [rows, col])
                acc = acc + uv * sv
            gbase = p * PB + g * L
            logits = acc + ub_v[pl.ds(gbase, L)] + sb_v[pl.ds(gbase, L)] + gb
            out_v[pl.ds(gbase, L)] = 1.0 / (1.0 + jnp.exp(-logits))
            return carry
        lax.fori_loop(0, PB // L, group, 0)

    bufs = [(urows0, srows0, sem0), (urows1, srows1, sem1)]
    fire_phase(0, *bufs[0])
    for c in bias_copies:
        c.wait()
    gb = gb_v[...]
    for p in range(NPHASE):
        if p + 1 < NPHASE:
            fire_phase(p + 1, *bufs[(p + 1) % 2])
        drain_phase(*bufs[p % 2])
        compute_phase(p, bufs[p % 2][0], bufs[p % 2][1], gb)

    pltpu.sync_copy(out_v, out_hbm.at[pl.ds(base, BPW)])


def kernel(u_ids, s_ids, u_embed, s_embed, u_bias, s_bias, global_bias):
    gb = jnp.broadcast_to(global_bias, (L,))
    return _cf_sc_kernel(u_ids, s_ids, u_embed, s_embed,
                         u_bias.reshape(-1), s_bias.reshape(-1), gb)


# speedup vs baseline: 1.4455x; 1.0034x over previous
import functools

import jax
import jax.numpy as jnp
from jax import lax
from jax.experimental import pallas as pl
from jax.experimental.pallas import tpu as pltpu
from jax.experimental.pallas import tpu_sc as plsc

B = 16384
D = 64
L = 16
NC = 2
NS = 16
NW = NC * NS
BPW = B // NW
CHUNK = 128
NCHUNK = BPW // CHUNK
PB = 128
NPHASE = BPW // PB

_mesh = plsc.VectorSubcoreMesh(core_axis_name="c", subcore_axis_name="s")


@functools.partial(
    pl.kernel,
    mesh=_mesh,
    out_type=jax.ShapeDtypeStruct((B,), jnp.float32),
    compiler_params=pltpu.CompilerParams(needs_layout_passes=False),
    scratch_types=[
        pltpu.VMEM((BPW,), jnp.int32),
        pltpu.VMEM((BPW,), jnp.int32),
        pltpu.VMEM((PB, D), jnp.float32),
        pltpu.VMEM((PB, D), jnp.float32),
        pltpu.VMEM((PB, D), jnp.float32),
        pltpu.VMEM((PB, D), jnp.float32),
        pltpu.VMEM((BPW,), jnp.float32),
        pltpu.VMEM((BPW,), jnp.float32),
        pltpu.VMEM((BPW,), jnp.float32),
        pltpu.VMEM((L,), jnp.float32),
        pltpu.SemaphoreType.DMA,
        pltpu.SemaphoreType.DMA,
        pltpu.SemaphoreType.DMA,
        pltpu.SemaphoreType.DMA,
        pltpu.SemaphoreType.DMA,
        pltpu.SemaphoreType.DMA,
        pltpu.SemaphoreType.DMA,
        pltpu.SemaphoreType.DMA,
        pltpu.SemaphoreType.DMA,
    ],
)
def _cf_sc_kernel(u_ids_hbm, s_ids_hbm, u_embed_hbm, s_embed_hbm,
                  u_bias_hbm, s_bias_hbm, gb_hbm, out_hbm,
                  uidx_v, sidx_v, urows0, urows1,
                  srows0, srows1, ub_v, sb_v, out_v, gb_v,
                  se0, se1, se2, se3, so0, so1, so2, so3, sem_bias):
    wid = lax.axis_index("s") * NC + lax.axis_index("c")
    base = wid * BPW

    pltpu.sync_copy(u_ids_hbm.at[pl.ds(base, BPW)], uidx_v)
    pltpu.sync_copy(s_ids_hbm.at[pl.ds(base, BPW)], sidx_v)
    pltpu.sync_copy(gb_hbm, gb_v)

    bias_copies = []
    for j in range(NCHUNK):
        dst = pl.ds(j * CHUNK, CHUNK)
        bias_copies.append(pltpu.async_copy(
            u_bias_hbm.at[uidx_v.at[dst]], ub_v.at[dst], sem_bias))
        bias_copies.append(pltpu.async_copy(
            s_bias_hbm.at[sidx_v.at[dst]], sb_v.at[dst], sem_bias))

    def fire_phase(p, ub, sb, sems):
        def body(g, carry):
            uvec = uidx_v[pl.ds(p * PB + g * L, L)]
            svec = sidx_v[pl.ds(p * PB + g * L, L)]
            for k in range(L):
                uid = uvec[k]
                sid = svec[k]
                i = g * L + k
                pltpu.async_copy(u_embed_hbm.at[pl.ds(uid, 1), :],
                                 ub.at[pl.ds(i, 1), :], sems[k % 4])
                pltpu.async_copy(s_embed_hbm.at[pl.ds(sid, 1), :],
                                 sb.at[pl.ds(i, 1), :], sems[(k + 1) % 4])
            return carry
        lax.fori_loop(0, PB // L, body, 0)

    def drain_phase(ub, sb, sems):
        for k in range(4):
            pltpu.make_async_copy(u_embed_hbm.at[pl.ds(0, PB // 4), :],
                                  ub.at[pl.ds(0, PB // 4), :], sems[k]).wait()
            pltpu.make_async_copy(u_embed_hbm.at[pl.ds(0, PB // 4), :],
                                  sb.at[pl.ds(0, PB // 4), :], sems[k]).wait()

    def compute_phase(p, ub, sb, gb):
        def group(g, carry):
            rows = g * L + lax.iota(jnp.int32, L)
            acc = jnp.zeros((L,), jnp.float32)
            for d in range(D):
                col = jnp.full((L,), d, jnp.int32)
                uv = plsc.load_gather(ub, [rows, col])
                sv = plsc.load_gather(sb, [rows, col])
                acc = acc + uv * sv
            gbase = p * PB + g * L
            logits = acc + ub_v[pl.ds(gbase, L)] + sb_v[pl.ds(gbase, L)] + gb
            out_v[pl.ds(gbase, L)] = 1.0 / (1.0 + jnp.exp(-logits))
            return carry
        lax.fori_loop(0, PB // L, group, 0)

    bufs = [(urows0, srows0, (se0, se1, se2, se3)),
            (urows1, srows1, (so0, so1, so2, so3))]
    fire_phase(0, *bufs[0])
    for c in bias_copies:
        c.wait()
    gb = gb_v[...]
    for p in range(NPHASE):
        if p + 1 < NPHASE:
            fire_phase(p + 1, *bufs[(p + 1) % 2])
        drain_phase(*bufs[p % 2])
        compute_phase(p, bufs[p % 2][0], bufs[p % 2][1], gb)

    pltpu.sync_copy(out_v, out_hbm.at[pl.ds(base, BPW)])


def kernel(u_ids, s_ids, u_embed, s_embed, u_bias, s_bias, global_bias):
    gb = jnp.broadcast_to(global_bias, (L,))
    return _cf_sc_kernel(u_ids, s_ids, u_embed, s_embed,
                         u_bias.reshape(-1), s_bias.reshape(-1), gb)
